# baseline (speedup 1.0000x reference)
import functools

import jax
import jax.numpy as jnp
from jax import lax
from jax.experimental import pallas as pl
from jax.experimental.pallas import tpu as pltpu
from jax.experimental.pallas import tpu_sc as plsc

D = 32
CH = 512
N_TOK = 4096 * 200


SPLIT = 251904


def _tc_row_major_table(w):
    w_t = jnp.transpose(w)

    rb = 2048

    def body(x0, x1, x2, x3, o_ref):
        for q, x in enumerate((x0, x1, x2, x3)):
            o_ref[:, q * 32 : (q + 1) * 32] = jnp.transpose(x[...])

    nb = SPLIT // rb
    return pl.pallas_call(
        body,
        grid=(nb,),
        in_specs=[
            pl.BlockSpec(
                (32, rb),
                functools.partial(
                    lambda q, i: (0, jnp.minimum(q * nb + i, 1000000 // rb)),
                    q,
                ),
            )
            for q in range(4)
        ],
        out_specs=pl.BlockSpec((rb, 128), lambda i: (i, 0)),
        out_shape=jax.ShapeDtypeStruct((SPLIT, 128), jnp.float32),
    )(w_t, w_t, w_t, w_t)


def _sc_gather_t(table, idx_flat, bias):
    info = plsc.get_sparse_core_info()
    nc, ns = info.num_cores, info.num_subcores
    nw = nc * ns
    n_ch = N_TOK // CH // nw
    mesh = plsc.VectorSubcoreMesh(core_axis_name="c", subcore_axis_name="s")

    @functools.partial(
        pl.kernel,
        mesh=mesh,
        out_type=jax.ShapeDtypeStruct((200, 4, 32, 8, 128), jnp.float32),
        compiler_params=pltpu.CompilerParams(
            use_tc_tiling_on_sc=False, needs_layout_passes=False
        ),
        scratch_types=[
            pltpu.VMEM((CH,), jnp.int32),
            pltpu.VMEM((CH,), jnp.int32),
            pltpu.VMEM((CH, D), jnp.float32),
            pltpu.VMEM((CH, D), jnp.float32),
            pltpu.VMEM((4, 4, 8, 128), jnp.float32),
            pltpu.VMEM((4, 4, 8, 128), jnp.float32),
            pltpu.VMEM((D, 16), jnp.float32),
            pltpu.SemaphoreType.DMA,
            pltpu.SemaphoreType.DMA,
            pltpu.SemaphoreType.DMA,
            pltpu.SemaphoreType.DMA,
            pltpu.SemaphoreType.DMA,
            pltpu.SemaphoreType.DMA,
        ],
    )
    def k(tbl_hbm, idx_hbm, bias_hbm, out_hbm,
          idx0, idx1, rows0, rows1, blk0, blk1, bias_v,
          is0, is1, gs0, gs1, os0, os1):
        wid = lax.axis_index("s") * nc + lax.axis_index("c")
        base = wid * n_ch
        idx_b = (idx0, idx1)
        rows_b = (rows0, rows1)
        blk_b = (blk0, blk1)
        is_b = (is0, is1)
        gs_b = (gs0, gs1)
        os_b = (os0, os1)
        pltpu.sync_copy(bias_hbm, bias_v)
        iota16 = lax.iota(jnp.int32, 16)
        z16 = jnp.zeros((16,), jnp.int32)

        pltpu.async_copy(idx_hbm.at[pl.ds(base * CH, CH)], idx0, is0)
        pltpu.async_copy(idx_hbm.at[pl.ds((base + 1) * CH, CH)], idx1, is1)
        pltpu.make_async_copy(idx_hbm.at[pl.ds(base * CH, CH)], idx0, is0).wait()
        pltpu.async_copy(tbl_hbm.at[idx0], rows0, gs0)

        def chunk_step(i, p):
            idx_v, rows_v, blk_v = idx_b[p], rows_b[p], blk_b[p]
            m = base + i
            pltpu.make_async_copy(tbl_hbm.at[idx_v], rows_v, gs_b[p]).wait()

            @pl.when(i + 1 < n_ch)
            def _():
                q = 1 - p
                pltpu.make_async_copy(
                    idx_hbm.at[pl.ds((m + 1) * CH, CH)], idx_b[q], is_b[q]
                ).wait()
                pltpu.async_copy(tbl_hbm.at[idx_b[q]], rows_b[q], gs_b[q])

            @pl.when(i + 2 < n_ch)
            def _():
                pltpu.async_copy(
                    idx_hbm.at[pl.ds((m + 2) * CH, CH)], idx_v, is_b[p]
                )

            @pl.when(i >= 2)
            def _():
                l_prev = (m - 2) // 8
                t_prev = ((m - 2) % 8) * 4
                pltpu.make_async_copy(
                    blk_v, out_hbm.at[l_prev, :, pl.ds(t_prev, 4)], os_b[p]
                ).wait()

            for c in range(D):
                bvec = bias_v[c, :]
                c16 = jnp.full((16,), c, jnp.int32)

                @plsc.parallel_loop(0, CH // 16, unroll=8)
                def _(g, c=c, bvec=bvec, c16=c16):
                    rvec = plsc.load_gather(rows_v, [g * 16 + iota16, c16])
                    blk_v[c // 8, g // 8, c % 8, pl.ds((g % 8) * 16, 16)] = (
                        rvec + bvec
                    )

            l_cur = m // 8
            t_cur = (m % 8) * 4
            pltpu.async_copy(
                blk_v, out_hbm.at[l_cur, :, pl.ds(t_cur, 4)], os_b[p]
            )

        def body(j, carry):
            chunk_step(2 * j, 0)
            chunk_step(2 * j + 1, 1)
            return carry

        lax.fori_loop(0, n_ch // 2, body, 0)

        for p in (0, 1):
            m = base + n_ch - 2 + p
            pltpu.make_async_copy(
                blk_b[p],
                out_hbm.at[m // 8, :, pl.ds((m % 8) * 4, 4)],
                os_b[p],
            ).wait()

    return k(table, idx_flat, bias)


def kernel(tokens, embedding_weight, embedding_bias):
    b, l = tokens.shape
    tok = tokens.astype(jnp.int32)
    tok = 4 * (tok % SPLIT) + tok // SPLIT
    idx = jnp.reshape(jnp.transpose(tok), (-1,))
    bias16 = jnp.broadcast_to(embedding_bias[:, None], (D, 16))
    tbl128 = _tc_row_major_table(embedding_weight)
    tbl = jnp.reshape(tbl128, (4 * SPLIT, D))
    out5 = _sc_gather_t(tbl, idx, bias16)
    return jnp.reshape(jnp.transpose(out5, (2, 4, 0, 1, 3)), (b, l, D))


# speedup vs baseline: 1.9139x; 1.9139x over previous
import functools

import jax
import jax.numpy as jnp
from jax import lax
from jax.experimental import pallas as pl
from jax.experimental.pallas import tpu as pltpu
from jax.experimental.pallas import tpu_sc as plsc

D = 32
CH = 512
N_TOK = 4096 * 200


SPLIT = 251904


def _tc_row_major_table(w):
    w_t = jnp.transpose(w)

    rb = 2048

    def body(x0, x1, x2, x3, o_ref):
        for q, x in enumerate((x0, x1, x2, x3)):
            o_ref[:, q * 32 : (q + 1) * 32] = jnp.transpose(x[...])

    nb = SPLIT // rb
    return pl.pallas_call(
        body,
        grid=(nb,),
        in_specs=[
            pl.BlockSpec(
                (32, rb),
                functools.partial(
                    lambda q, i: (0, jnp.minimum(q * nb + i, 1000000 // rb)),
                    q,
                ),
            )
            for q in range(4)
        ],
        out_specs=pl.BlockSpec((rb, 128), lambda i: (i, 0)),
        out_shape=jax.ShapeDtypeStruct((SPLIT, 128), jnp.float32),
    )(w_t, w_t, w_t, w_t)


def _sc_gather_t(table, idx_flat, bias):
    info = plsc.get_sparse_core_info()
    nc, ns = info.num_cores, info.num_subcores
    nw = nc * ns
    n_ch = N_TOK // CH // nw
    mesh = plsc.VectorSubcoreMesh(core_axis_name="c", subcore_axis_name="s")

    @functools.partial(
        pl.kernel,
        mesh=mesh,
        out_type=jax.ShapeDtypeStruct((200, 4, 32, 8, 128), jnp.float32),
        compiler_params=pltpu.CompilerParams(
            use_tc_tiling_on_sc=False, needs_layout_passes=False
        ),
        scratch_types=[
            pltpu.VMEM((CH,), jnp.int32),
            pltpu.VMEM((CH,), jnp.int32),
            pltpu.VMEM((CH, D), jnp.float32),
            pltpu.VMEM((CH, D), jnp.float32),
            pltpu.VMEM((4, 4, 8, 128), jnp.float32),
            pltpu.VMEM((4, 4, 8, 128), jnp.float32),
            pltpu.VMEM((D, 16), jnp.float32),
            pltpu.SemaphoreType.DMA,
            pltpu.SemaphoreType.DMA,
            pltpu.SemaphoreType.DMA,
            pltpu.SemaphoreType.DMA,
            pltpu.SemaphoreType.DMA,
            pltpu.SemaphoreType.DMA,
        ],
    )
    def k(tbl_hbm, idx_hbm, bias_hbm, out_hbm,
          idx0, idx1, rows0, rows1, blk0, blk1, bias_v,
          is0, is1, gs0, gs1, os0, os1):
        wid = lax.axis_index("s") * nc + lax.axis_index("c")
        base = wid * n_ch
        idx_b = (idx0, idx1)
        rows_b = (rows0, rows1)
        blk_b = (blk0, blk1)
        is_b = (is0, is1)
        gs_b = (gs0, gs1)
        os_b = (os0, os1)
        pltpu.sync_copy(bias_hbm, bias_v)
        iota16 = lax.iota(jnp.int32, 16)
        z16 = jnp.zeros((16,), jnp.int32)

        pltpu.async_copy(idx_hbm.at[pl.ds(base * CH, CH)], idx0, is0)
        pltpu.async_copy(idx_hbm.at[pl.ds((base + 1) * CH, CH)], idx1, is1)
        pltpu.make_async_copy(idx_hbm.at[pl.ds(base * CH, CH)], idx0, is0).wait()
        pltpu.async_copy(tbl_hbm.at[idx0], rows0, gs0)

        def chunk_step(i, p):
            idx_v, rows_v, blk_v = idx_b[p], rows_b[p], blk_b[p]
            m = base + i
            pltpu.make_async_copy(tbl_hbm.at[idx_v], rows_v, gs_b[p]).wait()

            @pl.when(i + 1 < n_ch)
            def _():
                q = 1 - p
                pltpu.make_async_copy(
                    idx_hbm.at[pl.ds((m + 1) * CH, CH)], idx_b[q], is_b[q]
                ).wait()
                pltpu.async_copy(tbl_hbm.at[idx_b[q]], rows_b[q], gs_b[q])

            @pl.when(i + 2 < n_ch)
            def _():
                pltpu.async_copy(
                    idx_hbm.at[pl.ds((m + 2) * CH, CH)], idx_v, is_b[p]
                )

            @pl.when(i >= 2)
            def _():
                l_prev = (m - 2) // 8
                t_prev = ((m - 2) % 8) * 4
                pltpu.make_async_copy(
                    blk_v, out_hbm.at[l_prev, :, pl.ds(t_prev, 4)], os_b[p]
                ).wait()

            for c in range(D):
                bvec = bias_v[c, :]
                fvec = (iota16 + c) & (D - 1)
                d0 = lax.shift_right_logical(fvec, 3)
                d2 = fvec & 7

                @plsc.parallel_loop(0, CH // 16, unroll=4)
                def _(g, bvec=bvec, fvec=fvec, d0=d0, d2=d2):
                    rvec = plsc.load_gather(rows_v, [g * 16 + iota16, fvec])
                    plsc.store_scatter(
                        blk_v,
                        [d0, z16 + g // 8, d2, (g % 8) * 16 + iota16],
                        rvec + bvec,
                    )

            l_cur = m // 8
            t_cur = (m % 8) * 4
            pltpu.async_copy(
                blk_v, out_hbm.at[l_cur, :, pl.ds(t_cur, 4)], os_b[p]
            )

        def body(j, carry):
            chunk_step(2 * j, 0)
            chunk_step(2 * j + 1, 1)
            return carry

        lax.fori_loop(0, n_ch // 2, body, 0)

        for p in (0, 1):
            m = base + n_ch - 2 + p
            pltpu.make_async_copy(
                blk_b[p],
                out_hbm.at[m // 8, :, pl.ds((m % 8) * 4, 4)],
                os_b[p],
            ).wait()

    return k(table, idx_flat, bias)


def kernel(tokens, embedding_weight, embedding_bias):
    b, l = tokens.shape
    tok = tokens.astype(jnp.int32)
    tok = 4 * (tok % SPLIT) + tok // SPLIT
    idx = jnp.reshape(jnp.transpose(tok), (-1,))
    c_idx = (jnp.arange(D)[:, None] + jnp.arange(16)[None, :]) % D
    bias16 = embedding_bias[c_idx]
    tbl128 = _tc_row_major_table(embedding_weight)
    tbl = jnp.reshape(tbl128, (4 * SPLIT, D))
    out5 = _sc_gather_t(tbl, idx, bias16)
    return jnp.reshape(jnp.transpose(out5, (2, 4, 0, 1, 3)), (b, l, D))


# speedup vs baseline: 1.9450x; 1.0162x over previous
import functools

import jax
import jax.numpy as jnp
from jax import lax
from jax.experimental import pallas as pl
from jax.experimental.pallas import tpu as pltpu
from jax.experimental.pallas import tpu_sc as plsc

D = 32
CH = 512
N_TOK = 4096 * 200


SPLIT = 253952


def _tc_row_major_table(w):
    w_t = jnp.transpose(w)

    rb = 4096

    def body(x0, x1, x2, x3, o_ref):
        for q, x in enumerate((x0, x1, x2, x3)):
            o_ref[:, q * 32 : (q + 1) * 32] = jnp.transpose(x[...])

    nb = SPLIT // rb
    return pl.pallas_call(
        body,
        grid=(nb,),
        in_specs=[
            pl.BlockSpec(
                (32, rb),
                functools.partial(
                    lambda q, i: (0, jnp.minimum(q * nb + i, 1000000 // rb)),
                    q,
                ),
            )
            for q in range(4)
        ],
        out_specs=pl.BlockSpec((rb, 128), lambda i: (i, 0)),
        out_shape=jax.ShapeDtypeStruct((SPLIT, 128), jnp.float32),
    )(w_t, w_t, w_t, w_t)


def _sc_gather_t(table, idx_flat, bias):
    info = plsc.get_sparse_core_info()
    nc, ns = info.num_cores, info.num_subcores
    nw = nc * ns
    n_ch = N_TOK // CH // nw
    mesh = plsc.VectorSubcoreMesh(core_axis_name="c", subcore_axis_name="s")

    @functools.partial(
        pl.kernel,
        mesh=mesh,
        out_type=jax.ShapeDtypeStruct((200, 4, 32, 8, 128), jnp.float32),
        compiler_params=pltpu.CompilerParams(
            use_tc_tiling_on_sc=False, needs_layout_passes=False
        ),
        scratch_types=[
            pltpu.VMEM((CH,), jnp.int32),
            pltpu.VMEM((CH,), jnp.int32),
            pltpu.VMEM((CH, D), jnp.float32),
            pltpu.VMEM((CH, D), jnp.float32),
            pltpu.VMEM((4, 4, 8, 128), jnp.float32),
            pltpu.VMEM((4, 4, 8, 128), jnp.float32),
            pltpu.VMEM((D, 16), jnp.float32),
            pltpu.SemaphoreType.DMA,
            pltpu.SemaphoreType.DMA,
            pltpu.SemaphoreType.DMA,
            pltpu.SemaphoreType.DMA,
            pltpu.SemaphoreType.DMA,
            pltpu.SemaphoreType.DMA,
        ],
    )
    def k(tbl_hbm, idx_hbm, bias_hbm, out_hbm,
          idx0, idx1, rows0, rows1, blk0, blk1, bias_v,
          is0, is1, gs0, gs1, os0, os1):
        wid = lax.axis_index("s") * nc + lax.axis_index("c")
        base = wid * n_ch
        idx_b = (idx0, idx1)
        rows_b = (rows0, rows1)
        blk_b = (blk0, blk1)
        is_b = (is0, is1)
        gs_b = (gs0, gs1)
        os_b = (os0, os1)
        pltpu.sync_copy(bias_hbm, bias_v)
        iota16 = lax.iota(jnp.int32, 16)
        z16 = jnp.zeros((16,), jnp.int32)

        pltpu.async_copy(idx_hbm.at[pl.ds(base * CH, CH)], idx0, is0)
        pltpu.async_copy(idx_hbm.at[pl.ds((base + 1) * CH, CH)], idx1, is1)
        pltpu.make_async_copy(idx_hbm.at[pl.ds(base * CH, CH)], idx0, is0).wait()
        pltpu.async_copy(tbl_hbm.at[idx0], rows0, gs0)

        def chunk_step(i, p):
            idx_v, rows_v, blk_v = idx_b[p], rows_b[p], blk_b[p]
            m = base + i
            pltpu.make_async_copy(tbl_hbm.at[idx_v], rows_v, gs_b[p]).wait()

            @pl.when(i + 1 < n_ch)
            def _():
                q = 1 - p
                pltpu.make_async_copy(
                    idx_hbm.at[pl.ds((m + 1) * CH, CH)], idx_b[q], is_b[q]
                ).wait()
                pltpu.async_copy(tbl_hbm.at[idx_b[q]], rows_b[q], gs_b[q])

            @pl.when(i + 2 < n_ch)
            def _():
                pltpu.async_copy(
                    idx_hbm.at[pl.ds((m + 2) * CH, CH)], idx_v, is_b[p]
                )

            @pl.when(i >= 2)
            def _():
                l_prev = (m - 2) // 8
                t_prev = ((m - 2) % 8) * 4
                pltpu.make_async_copy(
                    blk_v, out_hbm.at[l_prev, :, pl.ds(t_prev, 4)], os_b[p]
                ).wait()

            for c in range(D):
                bvec = bias_v[c, :]
                fvec = (iota16 + c) & (D - 1)
                d0 = lax.shift_right_logical(fvec, 3)
                d2 = fvec & 7

                @plsc.parallel_loop(0, CH // 16, unroll=4)
                def _(g, bvec=bvec, fvec=fvec, d0=d0, d2=d2):
                    rvec = plsc.load_gather(rows_v, [g * 16 + iota16, fvec])
                    plsc.store_scatter(
                        blk_v,
                        [d0, z16 + g // 8, d2, (g % 8) * 16 + iota16],
                        rvec + bvec,
                    )

            l_cur = m // 8
            t_cur = (m % 8) * 4
            pltpu.async_copy(
                blk_v, out_hbm.at[l_cur, :, pl.ds(t_cur, 4)], os_b[p]
            )

        def body(j, carry):
            chunk_step(2 * j, 0)
            chunk_step(2 * j + 1, 1)
            return carry

        lax.fori_loop(0, n_ch // 2, body, 0)

        for p in (0, 1):
            m = base + n_ch - 2 + p
            pltpu.make_async_copy(
                blk_b[p],
                out_hbm.at[m // 8, :, pl.ds((m % 8) * 4, 4)],
                os_b[p],
            ).wait()

    return k(table, idx_flat, bias)


def kernel(tokens, embedding_weight, embedding_bias):
    b, l = tokens.shape
    tok = tokens.astype(jnp.int32)
    tok = 4 * (tok % SPLIT) + tok // SPLIT
    idx = jnp.reshape(jnp.transpose(tok), (-1,))
    c_idx = (jnp.arange(D)[:, None] + jnp.arange(16)[None, :]) % D
    bias16 = embedding_bias[c_idx]
    tbl128 = _tc_row_major_table(embedding_weight)
    tbl = jnp.reshape(tbl128, (4 * SPLIT, D))
    out5 = _sc_gather_t(tbl, idx, bias16)
    return jnp.reshape(jnp.transpose(out5, (2, 4, 0, 1, 3)), (b, l, D))


# speedup vs baseline: 1.9684x; 1.0121x over previous
import functools

import jax
import jax.numpy as jnp
from jax import lax
from jax.experimental import pallas as pl
from jax.experimental.pallas import tpu as pltpu
from jax.experimental.pallas import tpu_sc as plsc

D = 32
CH = 512
N_TOK = 4096 * 200


SPLIT = 253952


def _tc_row_major_table(w):
    w_t = jnp.transpose(w)

    rb = 8192

    def body(x0, x1, x2, x3, o_ref):
        for q, x in enumerate((x0, x1, x2, x3)):
            o_ref[:, q * 32 : (q + 1) * 32] = jnp.transpose(x[...])

    nb = SPLIT // rb
    return pl.pallas_call(
        body,
        grid=(nb,),
        in_specs=[
            pl.BlockSpec(
                (32, rb),
                functools.partial(
                    lambda q, i: (0, jnp.minimum(q * nb + i, 1000000 // rb)),
                    q,
                ),
            )
            for q in range(4)
        ],
        out_specs=pl.BlockSpec((rb, 128), lambda i: (i, 0)),
        out_shape=jax.ShapeDtypeStruct((SPLIT, 128), jnp.float32),
    )(w_t, w_t, w_t, w_t)


def _sc_gather_t(table, idx_flat, bias):
    info = plsc.get_sparse_core_info()
    nc, ns = info.num_cores, info.num_subcores
    nw = nc * ns
    n_ch = N_TOK // CH // nw
    mesh = plsc.VectorSubcoreMesh(core_axis_name="c", subcore_axis_name="s")

    @functools.partial(
        pl.kernel,
        mesh=mesh,
        out_type=jax.ShapeDtypeStruct((200, 4, 32, 8, 128), jnp.float32),
        compiler_params=pltpu.CompilerParams(
            use_tc_tiling_on_sc=False, needs_layout_passes=False
        ),
        scratch_types=[
            pltpu.VMEM((CH,), jnp.int32),
            pltpu.VMEM((CH,), jnp.int32),
            pltpu.VMEM((CH, D), jnp.float32),
            pltpu.VMEM((CH, D), jnp.float32),
            pltpu.VMEM((4, 4, 8, 128), jnp.float32),
            pltpu.VMEM((4, 4, 8, 128), jnp.float32),
            pltpu.VMEM((D, 16), jnp.float32),
            pltpu.SemaphoreType.DMA,
            pltpu.SemaphoreType.DMA,
            pltpu.SemaphoreType.DMA,
            pltpu.SemaphoreType.DMA,
            pltpu.SemaphoreType.DMA,
            pltpu.SemaphoreType.DMA,
        ],
    )
    def k(tbl_hbm, idx_hbm, bias_hbm, out_hbm,
          idx0, idx1, rows0, rows1, blk0, blk1, bias_v,
          is0, is1, gs0, gs1, os0, os1):
        wid = lax.axis_index("s") * nc + lax.axis_index("c")
        base = wid * n_ch
        idx_b = (idx0, idx1)
        rows_b = (rows0, rows1)
        blk_b = (blk0, blk1)
        is_b = (is0, is1)
        gs_b = (gs0, gs1)
        os_b = (os0, os1)
        pltpu.sync_copy(bias_hbm, bias_v)
        iota16 = lax.iota(jnp.int32, 16)
        z16 = jnp.zeros((16,), jnp.int32)

        pltpu.async_copy(idx_hbm.at[pl.ds(base * CH, CH)], idx0, is0)
        pltpu.async_copy(idx_hbm.at[pl.ds((base + 1) * CH, CH)], idx1, is1)
        pltpu.make_async_copy(idx_hbm.at[pl.ds(base * CH, CH)], idx0, is0).wait()
        pltpu.async_copy(tbl_hbm.at[idx0], rows0, gs0)

        def chunk_step(i, p):
            idx_v, rows_v, blk_v = idx_b[p], rows_b[p], blk_b[p]
            m = base + i
            pltpu.make_async_copy(tbl_hbm.at[idx_v], rows_v, gs_b[p]).wait()

            @pl.when(i + 1 < n_ch)
            def _():
                q = 1 - p
                pltpu.make_async_copy(
                    idx_hbm.at[pl.ds((m + 1) * CH, CH)], idx_b[q], is_b[q]
                ).wait()
                pltpu.async_copy(tbl_hbm.at[idx_b[q]], rows_b[q], gs_b[q])

            @pl.when(i + 2 < n_ch)
            def _():
                pltpu.async_copy(
                    idx_hbm.at[pl.ds((m + 2) * CH, CH)], idx_v, is_b[p]
                )

            @pl.when(i >= 2)
            def _():
                l_prev = (m - 2) // 8
                t_prev = ((m - 2) % 8) * 4
                pltpu.make_async_copy(
                    blk_v, out_hbm.at[l_prev, :, pl.ds(t_prev, 4)], os_b[p]
                ).wait()

            for c in range(D):
                bvec = bias_v[c, :]
                fvec = (iota16 + c) & (D - 1)
                d0 = lax.shift_right_logical(fvec, 3)
                d2 = fvec & 7

                @plsc.parallel_loop(0, CH // 16, unroll=4)
                def _(g, bvec=bvec, fvec=fvec, d0=d0, d2=d2):
                    rvec = plsc.load_gather(rows_v, [g * 16 + iota16, fvec])
                    plsc.store_scatter(
                        blk_v,
                        [d0, z16 + g // 8, d2, (g % 8) * 16 + iota16],
                        rvec + bvec,
                    )

            l_cur = m // 8
            t_cur = (m % 8) * 4
            pltpu.async_copy(
                blk_v, out_hbm.at[l_cur, :, pl.ds(t_cur, 4)], os_b[p]
            )

        def body(j, carry):
            chunk_step(2 * j, 0)
            chunk_step(2 * j + 1, 1)
            return carry

        lax.fori_loop(0, n_ch // 2, body, 0)

        for p in (0, 1):
            m = base + n_ch - 2 + p
            pltpu.make_async_copy(
                blk_b[p],
                out_hbm.at[m // 8, :, pl.ds((m % 8) * 4, 4)],
                os_b[p],
            ).wait()

    return k(table, idx_flat, bias)


def kernel(tokens, embedding_weight, embedding_bias):
    b, l = tokens.shape
    tok = tokens.astype(jnp.int32)
    tok = 4 * (tok % SPLIT) + tok // SPLIT
    idx = jnp.reshape(jnp.transpose(tok), (-1,))
    c_idx = (jnp.arange(D)[:, None] + jnp.arange(16)[None, :]) % D
    bias16 = embedding_bias[c_idx]
    tbl128 = _tc_row_major_table(embedding_weight)
    tbl = jnp.reshape(tbl128, (4 * SPLIT, D))
    out5 = _sc_gather_t(tbl, idx, bias16)
    return jnp.reshape(jnp.transpose(out5, (2, 4, 0, 1, 3)), (b, l, D))
